# baseline (speedup 1.0000x reference)
import jax
import jax.numpy as jnp
from jax.experimental import pallas as pl
from jax.experimental.pallas import tpu as pltpu

N = 10000
F = 128
BM = 400
NB = N // BM
NBUF = 3


def _sage_body(adj_hbm, x_ref, w_ref, o_ref, abuf, sems):
    def copy_in(slot, idx):
        pltpu.make_async_copy(
            adj_hbm.at[pl.ds(idx * BM, BM), :],
            abuf.at[slot],
            sems.at[slot],
        ).start()

    for s in range(NBUF - 1):
        copy_in(s, s)

    xb = x_ref[...].astype(jnp.bfloat16)
    w1 = w_ref[:F, :]
    w2 = w_ref[F:, :]

    def step(i, carry):
        slot = jax.lax.rem(i, NBUF)
        nxt = i + NBUF - 1

        @pl.when(nxt < NB)
        def _prefetch():
            copy_in(jax.lax.rem(nxt, NBUF), nxt)

        pltpu.make_async_copy(
            adj_hbm.at[pl.ds(i * BM, BM), :],
            abuf.at[slot],
            sems.at[slot],
        ).wait()

        a = abuf[slot].astype(jnp.bfloat16)
        aggr = jnp.dot(a, xb, preferred_element_type=jnp.float32)
        xrow = x_ref[pl.ds(i * BM, BM), :]
        out = (
            jnp.dot(xrow, w1, preferred_element_type=jnp.float32)
            + jnp.dot(aggr, w2, preferred_element_type=jnp.float32)
        )
        o_ref[pl.ds(i * BM, BM), :] = jnp.maximum(out, 0.0)
        return carry

    jax.lax.fori_loop(0, NB, step, 0)


def kernel(x, adj, weight):
    return pl.pallas_call(
        _sage_body,
        in_specs=[
            pl.BlockSpec(memory_space=pltpu.HBM),
            pl.BlockSpec(memory_space=pltpu.VMEM),
            pl.BlockSpec(memory_space=pltpu.VMEM),
        ],
        out_specs=pl.BlockSpec(memory_space=pltpu.VMEM),
        out_shape=jax.ShapeDtypeStruct((N, F), jnp.float32),
        scratch_shapes=[
            pltpu.VMEM((NBUF, BM, N), jnp.float32),
            pltpu.SemaphoreType.DMA((NBUF,)),
        ],
        compiler_params=pltpu.CompilerParams(
            vmem_limit_bytes=100 * 1024 * 1024,
        ),
    )(adj, x, weight)


# speedup vs baseline: 1.0327x; 1.0327x over previous
import jax
import jax.numpy as jnp
from jax.experimental import pallas as pl
from jax.experimental.pallas import tpu as pltpu

N = 10000
F = 128
BM = 200


def _sage_step(adj0_ref, adj1_ref, x_ref, w_ref, o_ref):
    i = pl.program_id(0)
    xb = x_ref[...].astype(jnp.bfloat16)
    w1 = w_ref[:F, :]
    w2 = w_ref[F:, :]
    for h, a_ref in enumerate((adj0_ref, adj1_ref)):
        a = a_ref[...].astype(jnp.bfloat16)
        aggr = jnp.dot(a, xb, preferred_element_type=jnp.float32)
        xrow = x_ref[pl.ds(i * (2 * BM) + h * BM, BM), :]
        out = (
            jnp.dot(xrow, w1, preferred_element_type=jnp.float32)
            + jnp.dot(aggr, w2, preferred_element_type=jnp.float32)
        )
        o_ref[h * BM:(h + 1) * BM, :] = jnp.maximum(out, 0.0)


def kernel(x, adj, weight):
    grid = (N // (2 * BM),)
    return pl.pallas_call(
        _sage_step,
        grid=grid,
        in_specs=[
            pl.BlockSpec((BM, N), lambda i: (2 * i, 0)),
            pl.BlockSpec((BM, N), lambda i: (2 * i + 1, 0)),
            pl.BlockSpec((N, F), lambda i: (0, 0)),
            pl.BlockSpec((2 * F, F), lambda i: (0, 0)),
        ],
        out_specs=pl.BlockSpec((2 * BM, F), lambda i: (i, 0)),
        out_shape=jax.ShapeDtypeStruct((N, F), jnp.float32),
        compiler_params=pltpu.CompilerParams(
            dimension_semantics=("arbitrary",),
            vmem_limit_bytes=100 * 1024 * 1024,
        ),
    )(adj, adj, x, weight)


# speedup vs baseline: 1.0423x; 1.0094x over previous
import jax
import jax.numpy as jnp
from jax.experimental import pallas as pl
from jax.experimental.pallas import tpu as pltpu

N = 10000
F = 128
BM = 400


def _sage_step(adj_ref, x_ref, w_ref, o_ref):
    i = pl.program_id(0)
    a = adj_ref[...].astype(jnp.bfloat16)
    xb = x_ref[...].astype(jnp.bfloat16)
    aggr = jnp.dot(a, xb, preferred_element_type=jnp.float32)
    xrow = x_ref[pl.ds(i * BM, BM), :]
    out = (
        jnp.dot(xrow, w_ref[:F, :], preferred_element_type=jnp.float32)
        + jnp.dot(aggr, w_ref[F:, :], preferred_element_type=jnp.float32)
    )
    o_ref[...] = jnp.maximum(out, 0.0)


def kernel(x, adj, weight):
    grid = (N // BM,)
    return pl.pallas_call(
        _sage_step,
        grid=grid,
        in_specs=[
            pl.BlockSpec((BM, N), lambda i: (i, 0)),
            pl.BlockSpec((N, F), lambda i: (0, 0)),
            pl.BlockSpec((2 * F, F), lambda i: (0, 0)),
        ],
        out_specs=pl.BlockSpec((BM, F), lambda i: (i, 0)),
        out_shape=jax.ShapeDtypeStruct((N, F), jnp.float32),
        compiler_params=pltpu.CompilerParams(
            dimension_semantics=("arbitrary",),
            vmem_limit_bytes=100 * 1024 * 1024,
        ),
    )(adj, x, weight)
